# baseline (speedup 1.0000x reference)
import functools

import jax
import jax.numpy as jnp
import numpy as np
from jax import lax
from jax.experimental import pallas as pl
from jax.experimental.pallas import tpu as pltpu
from jax.experimental.pallas import tpu_sc as plsc

_B, _G = 16, 4096
_RATIO = 0.6
_LANES = 16


def _rotl32(x, d):
    return ((x << np.uint32(d)) | (x >> np.uint32(32 - d))).astype(np.uint32)


def _threefry2x32(ks0, ks1, x0, x1):
    rotations = ((13, 15, 26, 6), (17, 29, 16, 24))
    ks = (np.uint32(ks0), np.uint32(ks1),
          np.uint32(ks0) ^ np.uint32(ks1) ^ np.uint32(0x1BD11BDA))
    x = [(x0 + ks[0]).astype(np.uint32), (x1 + ks[1]).astype(np.uint32)]
    for i in range(5):
        for r in rotations[i % 2]:
            x[0] = (x[0] + x[1]).astype(np.uint32)
            x[1] = _rotl32(x[1], r) ^ x[0]
        x[0] = (x[0] + ks[(i + 1) % 3]).astype(np.uint32)
        x[1] = (x[1] + ks[(i + 2) % 3] + np.uint32(i + 1)).astype(np.uint32)
    return x


def _uniform_scores():
    n = _B * _G
    hi = np.zeros(n, dtype=np.uint32)
    lo = np.arange(n, dtype=np.uint32)
    o0, o1 = _threefry2x32(0, 42, hi, lo)
    bits = o0 ^ o1
    f = ((bits >> np.uint32(9)) | np.uint32(0x3F800000)).view(np.float32)
    f = np.maximum(np.float32(0.0), f - np.float32(1.0))
    return f.reshape(_B, _G)


def _build_tables():
    scores = _uniform_scores()
    order = np.argsort(scores, axis=1, kind="stable").astype(np.int32)
    rank = np.empty_like(order)
    rank[np.arange(_B)[:, None], order] = np.broadcast_to(
        np.arange(_G, dtype=np.int32)[None, :], (_B, _G))
    oa = (order.reshape(_B, _LANES, 2, _G // (2 * _LANES))
          .transpose(0, 3, 1, 2))
    orderp = (oa[..., 0] | (oa[..., 1] << 16)).reshape(-1)
    ra = rank.reshape(_B, _G // 32, 2, _LANES)
    rankp = (ra[:, :, 0, :] | (ra[:, :, 1, :] << 16)).reshape(-1)
    return orderp, rankp


_ORDERP, _RANKP = _build_tables()

_MESH = plsc.VectorSubcoreMesh(core_axis_name="c", subcore_axis_name="s")


@functools.partial(
    pl.kernel,
    out_type=(jax.ShapeDtypeStruct((_B, _G), jnp.int32),
              jax.ShapeDtypeStruct((_B, _G), jnp.int32)),
    mesh=_MESH,
    scratch_types=[
        pltpu.VMEM((_LANES,), jnp.int32),
        pltpu.VMEM((_G // 2,), jnp.int32),
        pltpu.VMEM((_G // 4,), jnp.int32),
        pltpu.VMEM((_G // 2,), jnp.int32),
        pltpu.VMEM((_G // 2,), jnp.int32),
        pltpu.SemaphoreType.DMA,
    ],
    compiler_params=pltpu.CompilerParams(needs_layout_passes=False),
)
def _mask_program(len_hbm, order_hbm, rank_hbm, m_hbm, nm_hbm,
                  len_v, order_v, rank_v, outm_v, outnm_v, sem):
    c = lax.axis_index("c")
    s = lax.axis_index("s")
    row = s

    rank_dma = pltpu.async_copy(
        rank_hbm.at[pl.ds(row * (_G // 2) + c * (_G // 4), _G // 4)],
        rank_v, sem)
    pltpu.sync_copy(len_hbm.at[pl.ds(0, _LANES)], len_v)
    pltpu.sync_copy(order_hbm.at[pl.ds(row * (_G // 2), _G // 2)], order_v)

    iota = lax.iota(jnp.int32, _LANES)
    lv = len_v[...]
    l_scal = jnp.sum(jnp.where(iota == row, lv, 0))
    l32 = jnp.full((_LANES,), l_scal, jnp.int32)
    nm32 = (l32.astype(jnp.float32) * jnp.float32(_RATIO)).astype(jnp.int32)

    zero32 = jnp.zeros((_LANES,), jnp.int32)
    nhalf = _G // (2 * _LANES)

    def _halves(x):
        return x & 0xFFFF, x >> 16

    def pass1(j, carry):
        cnt_a, cnt_b = carry
        a, b = _halves(order_v[pl.ds(j * _LANES, _LANES)])
        cnt_a = cnt_a + jnp.where(a < l32, 1, 0)
        cnt_b = cnt_b + jnp.where(b < l32, 1, 0)
        return cnt_a, cnt_b

    cnt_a, cnt_b = lax.fori_loop(0, nhalf, pass1, (zero32, zero32), unroll=4)

    blocktot = cnt_a + cnt_b
    exclp = plsc.cumsum(blocktot) - blocktot
    start_a = exclp
    start_b = exclp + cnt_a

    def pass2(j, carry):
        run_a, run_b, tacc = carry
        a, b = _halves(order_v[pl.ds(j * _LANES, _LANES)])
        run_a = run_a + jnp.where(a < l32, 1, 0)
        run_b = run_b + jnp.where(b < l32, 1, 0)
        tacc = (tacc + jnp.where(start_a + run_a <= nm32, 1, 0)
                + jnp.where(start_b + run_b <= nm32, 1, 0))
        return run_a, run_b, tacc

    _, _, tacc = lax.fori_loop(0, nhalf, pass2,
                               (zero32, zero32, zero32), unroll=4)
    tau32 = jnp.full((_LANES,), jnp.sum(tacc), jnp.int32)

    rank_dma.wait()

    halfpos = c * (_G // 2)

    def phase2(j, carry):
        vbase = j * (2 * _LANES)
        r_a, r_b = _halves(rank_v[pl.ds(j * _LANES, _LANES)])
        p_a = iota + (vbase + halfpos)
        p_b = p_a + _LANES
        va = p_a < l32
        vb = p_b < l32
        outm_v[pl.ds(vbase, _LANES)] = jnp.where(va & (r_a < tau32), 1, 0)
        outm_v[pl.ds(vbase + _LANES, _LANES)] = jnp.where(
            vb & (r_b < tau32), 1, 0)
        outnm_v[pl.ds(vbase, _LANES)] = jnp.where(va & (r_a >= tau32), 1, 0)
        outnm_v[pl.ds(vbase + _LANES, _LANES)] = jnp.where(
            vb & (r_b >= tau32), 1, 0)
        return carry

    lax.fori_loop(0, _G // (4 * _LANES), phase2, 0, unroll=4)

    halfg = c * (_G // 2)
    pltpu.sync_copy(outm_v, m_hbm.at[row, pl.ds(halfg, _G // 2)])
    pltpu.sync_copy(outnm_v, nm_hbm.at[row, pl.ds(halfg, _G // 2)])


def kernel(centers, lengths):
    del centers
    m_i32, nm_i32 = _mask_program(lengths, _ORDERP, _RANKP)
    return m_i32.astype(jnp.bool_), nm_i32.astype(jnp.bool_)
